# Initial kernel scaffold; baseline (speedup 1.0000x reference)
import jax
import jax.numpy as jnp
from jax.experimental import pallas as pl


def kernel(x, edge_index, gcn_W1, gcn_b1, gcn_W2, gcn_b2, u_W, u_b, r_W, r_b, c_W, c_b, out_W, out_b):
    raise NotImplementedError("write your pallas kernel here")


# speedup vs baseline: 12.3322x; 12.3322x over previous
import jax
import jax.numpy as jnp
from jax import lax
from jax.experimental import pallas as pl
from jax.experimental.pallas import tpu as pltpu
from jax.experimental.pallas import tpu_sc as plsc

_N = 10000
_E = 320000
_D = 128
_H = 128

_NC = 2
_NS = 16
_NW = _NC * _NS
_K = 80
_EPT = _E // _NW
_NCHUNK = _EPT // _K
_NPAD = 10240
_RPT = _NPAD // _NS

def _mesh():
    return plsc.VectorSubcoreMesh(
        core_axis_name="c", subcore_axis_name="s",
        num_cores=_NC, num_subcores=_NS)


def _deg_body(dst, zeros, out, idx_d, ones_v, acc):
    c = lax.axis_index("c")
    s = lax.axis_index("s")
    wid = c * _NS + s
    r0 = s * _RPT
    pltpu.sync_copy(zeros.at[pl.ds(r0, _RPT)], acc.at[pl.ds(r0, _RPT)])
    one = jnp.ones((16,), jnp.float32)

    def fill(r, carry):
        for k in range(8):
            ones_v[r, pl.ds(k * 16, 16)] = one
        return carry

    lax.fori_loop(0, _K, fill, 0)
    plsc.subcore_barrier()

    def step(j, carry):
        off = wid * _EPT + j * _K
        pltpu.sync_copy(dst.at[pl.ds(off, _K)], idx_d)
        pltpu.sync_copy(ones_v, acc.at[idx_d], add=True)
        return carry

    lax.fori_loop(0, _NCHUNK, step, 0)
    plsc.subcore_barrier()
    pltpu.sync_copy(acc.at[pl.ds(r0, _RPT)], out.at[c, pl.ds(r0, _RPT)])


def _sc_degree(dst, zeros):
    f = pl.kernel(
        _deg_body,
        out_type=jax.ShapeDtypeStruct((_NC, _NPAD, _D), jnp.float32),
        mesh=_mesh(),
        scratch_types=[
            pltpu.VMEM((_K,), jnp.int32),
            pltpu.VMEM((_K, _D), jnp.float32),
            pltpu.VMEM_SHARED((_NPAD, _D), jnp.float32),
        ],
    )
    return f(dst, zeros)


def _prop_body(z, src, dst, zeros, out, idx_s, idx_d, rows, acc, sem):
    c = lax.axis_index("c")
    s = lax.axis_index("s")
    wid = c * _NS + s
    r0 = s * _RPT
    pltpu.sync_copy(zeros.at[pl.ds(r0, _RPT)], acc.at[pl.ds(r0, _RPT)])
    plsc.subcore_barrier()

    def step(j, carry):
        off = wid * _EPT + j * _K
        pltpu.sync_copy(src.at[pl.ds(off, _K)], idx_s)
        pltpu.sync_copy(dst.at[pl.ds(off, _K)], idx_d)
        pltpu.async_copy(z.at[idx_s], rows, sem).wait()
        pltpu.sync_copy(rows, acc.at[idx_d], add=True)
        return carry

    lax.fori_loop(0, _NCHUNK, step, 0)
    plsc.subcore_barrier()
    pltpu.sync_copy(acc.at[pl.ds(r0, _RPT)], out.at[c, pl.ds(r0, _RPT)])


def _sc_prop(zarr, src, dst, zeros):
    f = pl.kernel(
        _prop_body,
        out_type=jax.ShapeDtypeStruct((_NC, _NPAD, _D), jnp.float32),
        mesh=_mesh(),
        scratch_types=[
            pltpu.VMEM((_K,), jnp.int32),
            pltpu.VMEM((_K,), jnp.int32),
            pltpu.VMEM((_K, _D), jnp.float32),
            pltpu.VMEM_SHARED((_NPAD, _D), jnp.float32),
            pltpu.SemaphoreType.DMA,
        ],
    )
    return f(zarr, src, dst, zeros)


_R = 2000
_G = _N // _R


def _pre_body(dpart, x, dinv_o, z1_o):
    indeg = dpart[0][:, 0:1] + dpart[1][:, 0:1]
    dinv = lax.rsqrt(indeg + 1.0)
    dinv_o[...] = dinv
    z1_o[...] = x[...] * dinv


def _tc_pre(dpart, x):
    return pl.pallas_call(
        _pre_body,
        grid=(_G,),
        in_specs=[
            pl.BlockSpec((_NC, _R, _D), lambda i: (0, i, 0)),
            pl.BlockSpec((_R, _D), lambda i: (i, 0)),
        ],
        out_specs=[
            pl.BlockSpec((_R, 1), lambda i: (i, 0)),
            pl.BlockSpec((_R, _D), lambda i: (i, 0)),
        ],
        out_shape=[
            jax.ShapeDtypeStruct((_N, 1), jnp.float32),
            jax.ShapeDtypeStruct((_N, _D), jnp.float32),
        ],
    )(dpart, x)


def _wfold_body(W2u, W2c, b2u, b2c, uW, cW, ub, cb, Wfu_o, Wfc_o, bfu_o, bfc_o):
    Wfu_o[...] = jnp.dot(W2u[...], uW[...], preferred_element_type=jnp.float32)
    Wfc_o[...] = jnp.dot(W2c[...], cW[...], preferred_element_type=jnp.float32)
    bfu_o[...] = jnp.dot(b2u[...], uW[...], preferred_element_type=jnp.float32) + ub[...]
    bfc_o[...] = jnp.dot(b2c[...], cW[...], preferred_element_type=jnp.float32) + cb[...]


def _tc_wfold(W2u, W2c, b2u, b2c, uW, cW, ub, cb):
    return pl.pallas_call(
        _wfold_body,
        out_shape=[
            jax.ShapeDtypeStruct((3 * _H, _H), jnp.float32),
            jax.ShapeDtypeStruct((3 * _H, _H), jnp.float32),
            jax.ShapeDtypeStruct((1, _H), jnp.float32),
            jax.ShapeDtypeStruct((1, _H), jnp.float32),
        ],
    )(W2u, W2c, b2u, b2c, uW, cW, ub, cb)


def _mid_body(s1, z1, dinv, W1, b1, Wfu, Wfc, zu_o, zc_o):
    dv = dinv[...]
    q = (s1[0] + s1[1] + z1[...]) * dv
    h1 = jnp.dot(q, W1[...], preferred_element_type=jnp.float32) + b1[...]
    h1 = jnp.maximum(h1, 0.0)
    zu_o[...] = jnp.dot(h1, Wfu[...], preferred_element_type=jnp.float32) * dv
    zc_o[...] = jnp.dot(h1, Wfc[...], preferred_element_type=jnp.float32) * dv


def _tc_mid(s1, z1, dinv, W1, b1, Wfu, Wfc):
    return pl.pallas_call(
        _mid_body,
        grid=(_G,),
        in_specs=[
            pl.BlockSpec((_NC, _R, _D), lambda i: (0, i, 0)),
            pl.BlockSpec((_R, _D), lambda i: (i, 0)),
            pl.BlockSpec((_R, 1), lambda i: (i, 0)),
            pl.BlockSpec((_D, 3 * _H), lambda i: (0, 0)),
            pl.BlockSpec((1, 3 * _H), lambda i: (0, 0)),
            pl.BlockSpec((3 * _H, _H), lambda i: (0, 0)),
            pl.BlockSpec((3 * _H, _H), lambda i: (0, 0)),
        ],
        out_specs=[
            pl.BlockSpec((_R, _H), lambda i: (i, 0)),
            pl.BlockSpec((_R, _H), lambda i: (i, 0)),
        ],
        out_shape=[
            jax.ShapeDtypeStruct((_N, _H), jnp.float32),
            jax.ShapeDtypeStruct((_N, _H), jnp.float32),
        ],
    )(s1, z1, dinv, W1, b1, Wfu, Wfc)


def _post_body(s2u, s2c, zu, zc, dinv, bfu, bfc, oW, ob, out_o, h_o):
    dv = dinv[...]
    pu = (s2u[0] + s2u[1] + zu[...]) * dv + bfu[...]
    pc = (s2c[0] + s2c[1] + zc[...]) * dv + bfc[...]
    u = jax.nn.sigmoid(pu)
    cg = jnp.tanh(pc)
    h = (1.0 - u) * cg
    h_o[...] = h
    out_o[...] = jnp.dot(h, oW[...], preferred_element_type=jnp.float32) + ob[...]


def _tc_post(s2u, s2c, zu, zc, dinv, bfu, bfc, oW, ob):
    return pl.pallas_call(
        _post_body,
        grid=(_G,),
        in_specs=[
            pl.BlockSpec((_NC, _R, _H), lambda i: (0, i, 0)),
            pl.BlockSpec((_NC, _R, _H), lambda i: (0, i, 0)),
            pl.BlockSpec((_R, _H), lambda i: (i, 0)),
            pl.BlockSpec((_R, _H), lambda i: (i, 0)),
            pl.BlockSpec((_R, 1), lambda i: (i, 0)),
            pl.BlockSpec((1, _H), lambda i: (0, 0)),
            pl.BlockSpec((1, _H), lambda i: (0, 0)),
            pl.BlockSpec((_H, 1), lambda i: (0, 0)),
            pl.BlockSpec((1, 1), lambda i: (0, 0)),
        ],
        out_specs=[
            pl.BlockSpec((_R, 1), lambda i: (i, 0)),
            pl.BlockSpec((_R, _H), lambda i: (i, 0)),
        ],
        out_shape=[
            jax.ShapeDtypeStruct((_N, 1), jnp.float32),
            jax.ShapeDtypeStruct((_N, _H), jnp.float32),
        ],
    )(s2u, s2c, zu, zc, dinv, bfu, bfc, oW, ob)


def kernel(x, edge_index, gcn_W1, gcn_b1, gcn_W2, gcn_b2,
           u_W, u_b, r_W, r_b, c_W, c_b, out_W, out_b):
    f32 = jnp.float32
    zeros128 = jnp.zeros((_NPAD, _D), f32)

    src = edge_index[0]
    dst = edge_index[1]
    dpart = _sc_degree(dst, zeros128)
    dinv, z1 = _tc_pre(dpart, x)
    s1 = _sc_prop(z1, src, dst, zeros128)

    Wfu, Wfc, bfu, bfc = _tc_wfold(
        gcn_W2[:, :_H], gcn_W2[:, 2 * _H:],
        gcn_b2[:_H].reshape(1, _H), gcn_b2[2 * _H:].reshape(1, _H),
        u_W[:_H], c_W[:_H],
        u_b.reshape(1, _H), c_b.reshape(1, _H),
    )
    zu, zc = _tc_mid(s1, z1, dinv, gcn_W1, gcn_b1.reshape(1, 3 * _H), Wfu, Wfc)

    s2u = _sc_prop(zu, src, dst, zeros128)
    s2c = _sc_prop(zc, src, dst, zeros128)

    out, h = _tc_post(s2u, s2c, zu, zc, dinv, bfu, bfc,
                      out_W, out_b.reshape(1, 1))
    return out, h
